# baseline (speedup 1.0000x reference)
import functools

import jax
import jax.numpy as jnp
from jax import lax
from jax.experimental import pallas as pl
from jax.experimental.pallas import tpu as pltpu
from jax.experimental.pallas import tpu_sc as plsc

NC = 2
NS = 16
NW = NC * NS
CB = 128
K = 4


def _sc_gather(table, ids_t, bsz, seq, d):
    mesh = plsc.VectorSubcoreMesh(core_axis_name="c", subcore_axis_name="s")
    bw = bsz // NW
    nbc = bw // CB
    nchunk = seq * nbc
    ngroups = nchunk // K
    dh, dl = d // 8, 8

    @functools.partial(
        pl.kernel,
        mesh=mesh,
        out_type=jax.ShapeDtypeStruct((seq, dh, bsz // CB, dl, CB), jnp.float32),
        scratch_types=[
            pltpu.VMEM((seq, bw), jnp.int32),
            pltpu.VMEM((K, CB, d), jnp.float32),
            pltpu.VMEM((K, CB, d), jnp.float32),
            pltpu.VMEM((K, dh, dl, CB), jnp.float32),
            pltpu.SemaphoreType.DMA,
            pltpu.SemaphoreType.DMA,
        ],
        compiler_params=pltpu.CompilerParams(
            use_tc_tiling_on_sc=False, needs_layout_passes=False
        ),
    )
    def k(table_hbm, ids_hbm, out5_hbm, idx_v, buf_a, buf_b, buf_t, gsem, ssem):
        wid = lax.axis_index("s") * NC + lax.axis_index("c")
        bbase = wid * bw
        pltpu.sync_copy(ids_hbm.at[:, pl.ds(bbase, bw)], idx_v)
        iota = lax.iota(jnp.int32, 16)

        def gather(j, buf, b):
            s = j // nbc
            b0 = (j - s * nbc) * CB
            pltpu.async_copy(table_hbm.at[idx_v.at[s, pl.ds(b0, CB)]], buf.at[b], gsem)

        def wait_gather(buf, b):
            pltpu.make_async_copy(
                table_hbm.at[idx_v.at[0, pl.ds(0, CB)]], buf.at[b], gsem
            ).wait()

        def transpose(buf, b):
            def body(l, carry):
                blv = l * 16 + iota
                for dd in range(d):
                    v = plsc.load_gather(
                        buf.at[b], [blv, jnp.full((16,), dd, jnp.int32)]
                    )
                    buf_t[b, dd // 8, dd % 8, pl.ds(l * 16, 16)] = v
                return carry

            lax.fori_loop(0, CB // 16, body, 0)

        def store(j, b):
            s = j // nbc
            bh = wid * nbc + (j - s * nbc)
            pltpu.async_copy(buf_t.at[b], out5_hbm.at[s, :, bh], ssem)

        def wait_store(b):
            pltpu.make_async_copy(buf_t.at[b], out5_hbm.at[0, :, 0], ssem).wait()

        for b in range(K):
            gather(b, buf_a, b)

        def half(g, cur, nxt):
            for b in range(K):
                wait_gather(cur, b)

            @pl.when(g + 1 < ngroups)
            def _():
                for b in range(K):
                    gather((g + 1) * K + b, nxt, b)

            for b in range(K):
                transpose(cur, b)
                store(g * K + b, b)
            for b in range(K):
                wait_store(b)

        def body(t, carry):
            half(2 * t, buf_a, buf_b)
            half(2 * t + 1, buf_b, buf_a)
            return carry

        lax.fori_loop(0, ngroups // 2, body, 0)

    return k(table, ids_t)


def kernel(input_ids, word_embeddings):
    bsz, seq = input_ids.shape
    _, d = word_embeddings.shape
    ids_t = input_ids.T.astype(jnp.int32)
    o = _sc_gather(word_embeddings, ids_t, bsz, seq, d)
    ot = jnp.transpose(o, (0, 1, 3, 2, 4)).reshape(seq, d, bsz)
    return jnp.transpose(ot, (2, 0, 1))


# speedup vs baseline: 1.4733x; 1.4733x over previous
import functools

import jax
import jax.numpy as jnp
from jax import lax
from jax.experimental import pallas as pl
from jax.experimental.pallas import tpu as pltpu
from jax.experimental.pallas import tpu_sc as plsc

NC = 2
NS = 16
NW = NC * NS
CB = 128
K = 4


def _sc_gather(table, ids_t, bsz, seq, d):
    mesh = plsc.VectorSubcoreMesh(core_axis_name="c", subcore_axis_name="s")
    bw = bsz // NW
    nbc = bw // CB
    nchunk = seq * nbc
    ngroups = nchunk // K

    @functools.partial(
        pl.kernel,
        mesh=mesh,
        out_type=jax.ShapeDtypeStruct((bsz, seq, d), jnp.float32),
        scratch_types=[
            pltpu.VMEM((seq, bw), jnp.int32),
            pltpu.VMEM((K, CB, d), jnp.float32),
            pltpu.VMEM((K, CB, d), jnp.float32),
            pltpu.SemaphoreType.DMA,
            pltpu.SemaphoreType.DMA,
        ],
        compiler_params=pltpu.CompilerParams(
            use_tc_tiling_on_sc=False, needs_layout_passes=False
        ),
    )
    def k(table_hbm, ids_hbm, out3_hbm, idx_v, buf_a, buf_b, gsem, ssem):
        wid = lax.axis_index("s") * NC + lax.axis_index("c")
        bbase = wid * bw
        pltpu.sync_copy(ids_hbm.at[:, pl.ds(bbase, bw)], idx_v)

        def gather(j, buf, b):
            s = j // nbc
            b0 = (j - s * nbc) * CB
            pltpu.async_copy(table_hbm.at[idx_v.at[s, pl.ds(b0, CB)]], buf.at[b], gsem)

        def wait_gather(buf, b):
            pltpu.make_async_copy(
                table_hbm.at[idx_v.at[0, pl.ds(0, CB)]], buf.at[b], gsem
            ).wait()

        def store(j, buf, b):
            s = j // nbc
            b0 = (j - s * nbc) * CB
            pltpu.async_copy(
                buf.at[b], out3_hbm.at[pl.ds(bbase + b0, CB), s], ssem
            )

        def wait_store(buf, b):
            pltpu.make_async_copy(
                buf.at[b], out3_hbm.at[pl.ds(bbase, CB), 0], ssem
            ).wait()

        for b in range(K):
            gather(b, buf_a, b)

        def half(g, cur, nxt):
            for b in range(K):
                wait_gather(cur, b)

            @pl.when(g + 1 < ngroups)
            def _():
                for b in range(K):
                    gather((g + 1) * K + b, nxt, b)

            for b in range(K):
                store(g * K + b, cur, b)
            for b in range(K):
                wait_store(cur, b)

        def body(t, carry):
            half(2 * t, buf_a, buf_b)
            half(2 * t + 1, buf_b, buf_a)
            return carry

        lax.fori_loop(0, ngroups // 2, body, 0)

    return k(table, ids_t)


def kernel(input_ids, word_embeddings):
    bsz, seq = input_ids.shape
    _, d = word_embeddings.shape
    ids_t = input_ids.T.astype(jnp.int32)
    return _sc_gather(word_embeddings, ids_t, bsz, seq, d)


# speedup vs baseline: 1.7748x; 1.2046x over previous
import functools

import jax
import jax.numpy as jnp
from jax import lax
from jax.experimental import pallas as pl
from jax.experimental.pallas import tpu as pltpu
from jax.experimental.pallas import tpu_sc as plsc

NC = 2
NS = 16
NW = NC * NS
CB = 128
K = 4


def _sc_gather(table, ids_t, bsz, seq, d):
    mesh = plsc.VectorSubcoreMesh(core_axis_name="c", subcore_axis_name="s")
    bw = bsz // NW
    nbc = bw // CB
    nchunk = seq * nbc
    ngroups = nchunk // K

    @functools.partial(
        pl.kernel,
        mesh=mesh,
        out_type=jax.ShapeDtypeStruct((seq, d // 8, bsz // CB, 8, CB), jnp.float32),
        scratch_types=[
            pltpu.VMEM((seq, bw), jnp.int32),
            pltpu.VMEM((K, CB, d), jnp.float32),
            pltpu.VMEM((K, CB, d), jnp.float32),
            pltpu.VMEM((K, d // 8, 8, CB), jnp.float32),
            pltpu.SemaphoreType.DMA,
            pltpu.SemaphoreType.DMA,
        ],
        compiler_params=pltpu.CompilerParams(
            use_tc_tiling_on_sc=False, needs_layout_passes=False
        ),
    )
    def k(table_hbm, ids_hbm, out5_hbm, idx_v, buf_a, buf_b, buf_t, gsem, ssem):
        wid = lax.axis_index("s") * NC + lax.axis_index("c")
        bbase = wid * bw
        pltpu.sync_copy(ids_hbm.at[:, pl.ds(bbase, bw)], idx_v)
        iota = lax.iota(jnp.int32, 16)
        rowv = [iota + 16 * t for t in range(CB // 16)]

        def gather(j, buf, b):
            s = j // nbc
            b0 = (j - s * nbc) * CB
            pltpu.async_copy(table_hbm.at[idx_v.at[s, pl.ds(b0, CB)]], buf.at[b], gsem)

        def wait_gather(buf, b):
            pltpu.make_async_copy(
                table_hbm.at[idx_v.at[0, pl.ds(0, CB)]], buf.at[b], gsem
            ).wait()

        def transpose(buf, b):
            src = buf.at[b]
            dst = buf_t.at[b]

            def body(j, carry):
                rot = jnp.bitwise_and(iota + j, 15)
                for t in range(CB // 16):
                    for q in range(d // 16):
                        dv = rot + (16 * q)
                        v = plsc.load_gather(src, [rowv[t], dv])
                        plsc.store_scatter(
                            dst,
                            [
                                lax.shift_right_logical(dv, 3),
                                jnp.bitwise_and(dv, 7),
                                rowv[t],
                            ],
                            v,
                        )
                return carry

            lax.fori_loop(0, 16, body, 0)

        def store(j, b):
            s = j // nbc
            bh = wid * nbc + (j - s * nbc)
            pltpu.async_copy(buf_t.at[b], out5_hbm.at[s, :, bh], ssem)

        def wait_store(b):
            pltpu.make_async_copy(buf_t.at[b], out5_hbm.at[0, :, 0], ssem).wait()

        for b in range(K):
            gather(b, buf_a, b)

        def half(g, cur, nxt):
            for b in range(K):
                wait_gather(cur, b)

            @pl.when(g + 1 < ngroups)
            def _():
                for b in range(K):
                    gather((g + 1) * K + b, nxt, b)

            for b in range(K):
                transpose(cur, b)
                store(g * K + b, b)
            for b in range(K):
                wait_store(b)

        def body(t, carry):
            half(2 * t, buf_a, buf_b)
            half(2 * t + 1, buf_b, buf_a)
            return carry

        lax.fori_loop(0, ngroups // 2, body, 0)

    return k(table, ids_t)


def kernel(input_ids, word_embeddings):
    bsz, seq = input_ids.shape
    _, d = word_embeddings.shape
    ids_t = input_ids.T.astype(jnp.int32)
    o = _sc_gather(word_embeddings, ids_t, bsz, seq, d)
    ot = jnp.transpose(o, (0, 1, 3, 2, 4)).reshape(seq, d, bsz)
    return jnp.transpose(ot, (2, 0, 1))


# speedup vs baseline: 1.7776x; 1.0016x over previous
import functools

import jax
import jax.numpy as jnp
from jax import lax
from jax.experimental import pallas as pl
from jax.experimental.pallas import tpu as pltpu
from jax.experimental.pallas import tpu_sc as plsc

NC = 2
NS = 16
NW = NC * NS
CB = 128
K = 4


def _sc_gather(table, ids_t, bsz, seq, d):
    mesh = plsc.VectorSubcoreMesh(core_axis_name="c", subcore_axis_name="s")
    bw = bsz // NW
    nbc = bw // CB
    nchunk = seq * nbc
    ngroups = nchunk // K

    @functools.partial(
        pl.kernel,
        mesh=mesh,
        out_type=jax.ShapeDtypeStruct((seq, d // 8, bsz // CB, 8, CB), jnp.float32),
        scratch_types=[
            pltpu.VMEM((seq, bw), jnp.int32),
            pltpu.VMEM((K, CB, d), jnp.float32),
            pltpu.VMEM((K, CB, d), jnp.float32),
            pltpu.VMEM((K, d // 8, 8, CB), jnp.float32),
            pltpu.SemaphoreType.DMA,
            pltpu.SemaphoreType.DMA,
        ],
        compiler_params=pltpu.CompilerParams(
            use_tc_tiling_on_sc=False, needs_layout_passes=False
        ),
    )
    def k(table_hbm, ids_hbm, out5_hbm, idx_v, buf_a, buf_b, buf_t, gsem, ssem):
        wid = lax.axis_index("s") * NC + lax.axis_index("c")
        bbase = wid * bw
        pltpu.sync_copy(ids_hbm.at[:, pl.ds(bbase, bw)], idx_v)
        iota = lax.iota(jnp.int32, 16)
        rowv = [iota + 16 * t for t in range(CB // 16)]

        def gather(j, buf, b):
            s = j // nbc
            b0 = (j - s * nbc) * CB
            pltpu.async_copy(table_hbm.at[idx_v.at[s, pl.ds(b0, CB)]], buf.at[b], gsem)

        def wait_gather(buf, b):
            pltpu.make_async_copy(
                table_hbm.at[idx_v.at[0, pl.ds(0, CB)]], buf.at[b], gsem
            ).wait()

        def transpose(buf, b):
            src = buf.at[b]
            dst = buf_t.at[b]

            def body(j, carry):
                rot = jnp.bitwise_and(iota + j, 15)
                dl7 = jnp.bitwise_and(rot, 7)
                rot3 = lax.shift_right_logical(rot, 3)
                dhq = [rot3 + 2 * q for q in range(d // 16)]
                for t in range(CB // 16):
                    for q in range(d // 16):
                        v = plsc.load_gather(src, [rowv[t], rot + (16 * q)])
                        plsc.store_scatter(dst, [dhq[q], dl7, rowv[t]], v)
                return carry

            lax.fori_loop(0, 16, body, 0)

        def store(j, b):
            s = j // nbc
            bh = wid * nbc + (j - s * nbc)
            pltpu.async_copy(buf_t.at[b], out5_hbm.at[s, :, bh], ssem)

        def wait_store(b):
            pltpu.make_async_copy(buf_t.at[b], out5_hbm.at[0, :, 0], ssem).wait()

        for b in range(K):
            gather(b, buf_a, b)

        def half(g, cur, nxt):
            for b in range(K):
                wait_gather(cur, b)

            @pl.when(g + 1 < ngroups)
            def _():
                for b in range(K):
                    gather((g + 1) * K + b, nxt, b)

            for b in range(K):
                transpose(cur, b)
                store(g * K + b, b)
            for b in range(K):
                wait_store(b)

        def body(t, carry):
            half(2 * t, buf_a, buf_b)
            half(2 * t + 1, buf_b, buf_a)
            return carry

        lax.fori_loop(0, ngroups // 2, body, 0)

    return k(table, ids_t)


def kernel(input_ids, word_embeddings):
    bsz, seq = input_ids.shape
    _, d = word_embeddings.shape
    ids_t = input_ids.T.astype(jnp.int32)
    o = _sc_gather(word_embeddings, ids_t, bsz, seq, d)
    ot = jnp.transpose(o, (0, 1, 3, 2, 4)).reshape(seq, d, bsz)
    return jnp.transpose(ot, (2, 0, 1))
